# baseline (speedup 1.0000x reference)
import functools

import jax
import jax.numpy as jnp
from jax.experimental import pallas as pl
from jax.experimental.pallas import tpu as pltpu

NCLS = 81
BN_EPS = 1e-3
NEG = -1e30


def _gemm1_body(a_ref, w_ref, b_ref, o_ref, s_ref, q_ref, *, mt):
    h = pl.program_id(0)
    m = pl.program_id(1)
    kr = w_ref.shape[1]

    a = a_ref[...].reshape(mt, kr).astype(jnp.bfloat16)
    wt = w_ref[0].astype(jnp.bfloat16)
    y = jnp.dot(a, wt, preferred_element_type=jnp.float32)

    row = pl.ds(pl.multiple_of(m * mt, 8), mt)

    @pl.when(h == 0)
    def _init():
        o_ref[row, :] = y

    @pl.when(h > 0)
    def _acc():
        o_ref[row, :] += y

    @pl.when((h == 6) & (m == pl.num_programs(1) - 1))
    def _fini():
        x = o_ref[...] + b_ref[...]
        o_ref[...] = x
        s_ref[...] = jnp.sum(x, axis=0, keepdims=True)
        q_ref[...] = jnp.sum(x * x, axis=0, keepdims=True)


def _mid_body(x_ref, s_ref, q_ref, g_ref, bt_ref, w_ref, b_ref,
              o_ref, s2_ref, q2_ref, *, n):
    inv_n = 1.0 / n
    mean = s_ref[...] * inv_n
    var = q_ref[...] * inv_n - mean * mean
    scale = g_ref[...] * jax.lax.rsqrt(var + BN_EPS)
    xn = jnp.maximum((x_ref[...] - mean) * scale + bt_ref[...], 0.0)
    y = jnp.dot(xn.astype(jnp.bfloat16), w_ref[...].astype(jnp.bfloat16),
                preferred_element_type=jnp.float32) + b_ref[...]
    o_ref[...] = y

    @pl.when(pl.program_id(0) == 0)
    def _zero():
        s2_ref[...] = jnp.zeros_like(s2_ref)
        q2_ref[...] = jnp.zeros_like(q2_ref)

    s2_ref[...] += jnp.sum(y, axis=0, keepdims=True)
    q2_ref[...] += jnp.sum(y * y, axis=0, keepdims=True)


def _head_body(x_ref, s_ref, q_ref, g_ref, bt_ref, wl_ref, bl_ref,
               wd_ref, bd_ref, lo_ref, po_ref, do_ref, *, n):
    inv_n = 1.0 / n
    mean = s_ref[...] * inv_n
    var = q_ref[...] * inv_n - mean * mean
    scale = g_ref[...] * jax.lax.rsqrt(var + BN_EPS)
    xn = jnp.maximum((x_ref[...] - mean) * scale + bt_ref[...], 0.0)
    xb = xn.astype(jnp.bfloat16)

    logits = jnp.dot(xb, wl_ref[...].astype(jnp.bfloat16),
                     preferred_element_type=jnp.float32) + bl_ref[...]
    lo_ref[...] = logits
    m = jnp.max(logits, axis=-1, keepdims=True)
    e = jnp.exp(logits - m)
    po_ref[...] = e / jnp.sum(e, axis=-1, keepdims=True)

    do_ref[...] = jnp.dot(xb, wd_ref[...].astype(jnp.bfloat16),
                          preferred_element_type=jnp.float32) + bd_ref[...]


def kernel(pooled_rois, W1, b1, gamma1, beta1, W2, b2, gamma2, beta2, Wl, bl, Wd, bd):
    N = pooled_rois.shape[0]
    K = 7 * 7 * pooled_rois.shape[3]
    C = W1.shape[3]

    W2r = W2.reshape(C, C)

    LP = 128
    DP = 384
    Wl_p = jnp.zeros((C, LP), W1.dtype).at[:, :NCLS].set(Wl)
    bl_p = jnp.full((1, LP), NEG, W1.dtype).at[0, :NCLS].set(bl)
    Wd_p = jnp.zeros((C, DP), W1.dtype).at[:, :4 * NCLS].set(Wd)
    bd_p = jnp.zeros((1, DP), W1.dtype).at[0, :4 * NCLS].set(bd)

    b1r = b1.reshape(1, C)
    b2r = b2.reshape(1, C)
    g1r = gamma1.reshape(1, C)
    bt1r = beta1.reshape(1, C)
    g2r = gamma2.reshape(1, C)
    bt2r = beta2.reshape(1, C)

    CH = pooled_rois.shape[3]
    KR = 7 * CH
    W1r = W1.reshape(7, KR, C)
    MT1 = 400
    x1, s1, q1 = pl.pallas_call(
        functools.partial(_gemm1_body, mt=MT1),
        grid=(7, N // MT1),
        in_specs=[
            pl.BlockSpec((MT1, 1, 7, CH), lambda h, m: (m, h, 0, 0)),
            pl.BlockSpec((1, KR, C), lambda h, m: (h, 0, 0)),
            pl.BlockSpec((1, C), lambda h, m: (0, 0)),
        ],
        out_specs=[
            pl.BlockSpec((N, C), lambda h, m: (0, 0)),
            pl.BlockSpec((1, C), lambda h, m: (0, 0)),
            pl.BlockSpec((1, C), lambda h, m: (0, 0)),
        ],
        out_shape=[
            jax.ShapeDtypeStruct((N, C), jnp.float32),
            jax.ShapeDtypeStruct((1, C), jnp.float32),
            jax.ShapeDtypeStruct((1, C), jnp.float32),
        ],
        compiler_params=pltpu.CompilerParams(
            dimension_semantics=("arbitrary", "arbitrary")),
    )(pooled_rois, W1r, b1r)

    MT2 = 400
    x2, s2, q2 = pl.pallas_call(
        functools.partial(_mid_body, n=float(N)),
        grid=(N // MT2,),
        in_specs=[
            pl.BlockSpec((MT2, C), lambda m: (m, 0)),
            pl.BlockSpec((1, C), lambda m: (0, 0)),
            pl.BlockSpec((1, C), lambda m: (0, 0)),
            pl.BlockSpec((1, C), lambda m: (0, 0)),
            pl.BlockSpec((1, C), lambda m: (0, 0)),
            pl.BlockSpec((C, C), lambda m: (0, 0)),
            pl.BlockSpec((1, C), lambda m: (0, 0)),
        ],
        out_specs=[
            pl.BlockSpec((MT2, C), lambda m: (m, 0)),
            pl.BlockSpec((1, C), lambda m: (0, 0)),
            pl.BlockSpec((1, C), lambda m: (0, 0)),
        ],
        out_shape=[
            jax.ShapeDtypeStruct((N, C), jnp.float32),
            jax.ShapeDtypeStruct((1, C), jnp.float32),
            jax.ShapeDtypeStruct((1, C), jnp.float32),
        ],
        compiler_params=pltpu.CompilerParams(
            dimension_semantics=("arbitrary",)),
    )(x1, s1, q1, g1r, bt1r, W2r, b2r)

    lo, po, do_ = pl.pallas_call(
        functools.partial(_head_body, n=float(N)),
        grid=(N // MT2,),
        in_specs=[
            pl.BlockSpec((MT2, C), lambda m: (m, 0)),
            pl.BlockSpec((1, C), lambda m: (0, 0)),
            pl.BlockSpec((1, C), lambda m: (0, 0)),
            pl.BlockSpec((1, C), lambda m: (0, 0)),
            pl.BlockSpec((1, C), lambda m: (0, 0)),
            pl.BlockSpec((C, LP), lambda m: (0, 0)),
            pl.BlockSpec((1, LP), lambda m: (0, 0)),
            pl.BlockSpec((C, DP), lambda m: (0, 0)),
            pl.BlockSpec((1, DP), lambda m: (0, 0)),
        ],
        out_specs=[
            pl.BlockSpec((MT2, LP), lambda m: (m, 0)),
            pl.BlockSpec((MT2, LP), lambda m: (m, 0)),
            pl.BlockSpec((MT2, DP), lambda m: (m, 0)),
        ],
        out_shape=[
            jax.ShapeDtypeStruct((N, LP), jnp.float32),
            jax.ShapeDtypeStruct((N, LP), jnp.float32),
            jax.ShapeDtypeStruct((N, DP), jnp.float32),
        ],
        compiler_params=pltpu.CompilerParams(
            dimension_semantics=("arbitrary",)),
    )(x2, s2, q2, g2r, bt2r, Wl_p, bl_p, Wd_p, bd_p)

    logits = lo[:, :NCLS]
    probs = po[:, :NCLS]
    deltas = do_[:, :4 * NCLS].reshape(N, NCLS, 4)
    return (logits, probs, deltas)


# speedup vs baseline: 1.0119x; 1.0119x over previous
import functools

import jax
import jax.numpy as jnp
from jax.experimental import pallas as pl
from jax.experimental.pallas import tpu as pltpu

NCLS = 81
BN_EPS = 1e-3
NEG = -1e30


def _gemm1_body(a_ref, w_ref, b_ref, o_ref, s_ref, q_ref, acc, *, mt):
    h = pl.program_id(0)
    m = pl.program_id(1)
    kr = w_ref.shape[1]

    a = a_ref[...].reshape(mt, kr).astype(jnp.bfloat16)
    wt = w_ref[0].astype(jnp.bfloat16)
    y = jnp.dot(a, wt, preferred_element_type=jnp.float32)

    row = pl.ds(pl.multiple_of(m * mt, 8), mt)

    @pl.when(h == 0)
    def _init():
        acc[row, :] = y

    @pl.when(h > 0)
    def _acc():
        acc[row, :] += y

    @pl.when(h == 6)
    def _fini():
        x = acc[row, :] + b_ref[...]
        o_ref[...] = x

        @pl.when(m == 0)
        def _zero():
            s_ref[...] = jnp.zeros_like(s_ref)
            q_ref[...] = jnp.zeros_like(q_ref)

        s_ref[...] += jnp.sum(x, axis=0, keepdims=True)
        q_ref[...] += jnp.sum(x * x, axis=0, keepdims=True)


def _mid_body(x_ref, s_ref, q_ref, g_ref, bt_ref, w_ref, b_ref,
              o_ref, s2_ref, q2_ref, *, n):
    inv_n = 1.0 / n
    mean = s_ref[...] * inv_n
    var = q_ref[...] * inv_n - mean * mean
    scale = g_ref[...] * jax.lax.rsqrt(var + BN_EPS)
    xn = jnp.maximum((x_ref[...] - mean) * scale + bt_ref[...], 0.0)
    y = jnp.dot(xn.astype(jnp.bfloat16), w_ref[...].astype(jnp.bfloat16),
                preferred_element_type=jnp.float32) + b_ref[...]
    o_ref[...] = y

    @pl.when(pl.program_id(0) == 0)
    def _zero():
        s2_ref[...] = jnp.zeros_like(s2_ref)
        q2_ref[...] = jnp.zeros_like(q2_ref)

    s2_ref[...] += jnp.sum(y, axis=0, keepdims=True)
    q2_ref[...] += jnp.sum(y * y, axis=0, keepdims=True)


def _head_body(x_ref, s_ref, q_ref, g_ref, bt_ref, wl_ref, bl_ref,
               wd_ref, bd_ref, lo_ref, po_ref, do_ref, *, n):
    inv_n = 1.0 / n
    mean = s_ref[...] * inv_n
    var = q_ref[...] * inv_n - mean * mean
    scale = g_ref[...] * jax.lax.rsqrt(var + BN_EPS)
    xn = jnp.maximum((x_ref[...] - mean) * scale + bt_ref[...], 0.0)
    xb = xn.astype(jnp.bfloat16)

    logits = jnp.dot(xb, wl_ref[...].astype(jnp.bfloat16),
                     preferred_element_type=jnp.float32) + bl_ref[...]
    lo_ref[...] = logits
    m = jnp.max(logits, axis=-1, keepdims=True)
    e = jnp.exp(logits - m)
    po_ref[...] = e / jnp.sum(e, axis=-1, keepdims=True)

    do_ref[...] = jnp.dot(xb, wd_ref[...].astype(jnp.bfloat16),
                          preferred_element_type=jnp.float32) + bd_ref[...]


def kernel(pooled_rois, W1, b1, gamma1, beta1, W2, b2, gamma2, beta2, Wl, bl, Wd, bd):
    N = pooled_rois.shape[0]
    K = 7 * 7 * pooled_rois.shape[3]
    C = W1.shape[3]

    W2r = W2.reshape(C, C)

    LP = 128
    DP = 384
    Wl_p = jnp.zeros((C, LP), W1.dtype).at[:, :NCLS].set(Wl)
    bl_p = jnp.full((1, LP), NEG, W1.dtype).at[0, :NCLS].set(bl)
    Wd_p = jnp.zeros((C, DP), W1.dtype).at[:, :4 * NCLS].set(Wd)
    bd_p = jnp.zeros((1, DP), W1.dtype).at[0, :4 * NCLS].set(bd)

    b1r = b1.reshape(1, C)
    b2r = b2.reshape(1, C)
    g1r = gamma1.reshape(1, C)
    bt1r = beta1.reshape(1, C)
    g2r = gamma2.reshape(1, C)
    bt2r = beta2.reshape(1, C)

    CH = pooled_rois.shape[3]
    KR = 7 * CH
    W1r = W1.reshape(7, KR, C)
    MT1 = 1000
    x1, s1, q1 = pl.pallas_call(
        functools.partial(_gemm1_body, mt=MT1),
        grid=(7, N // MT1),
        in_specs=[
            pl.BlockSpec((MT1, 1, 7, CH), lambda h, m: (m, h, 0, 0)),
            pl.BlockSpec((1, KR, C), lambda h, m: (h, 0, 0)),
            pl.BlockSpec((1, C), lambda h, m: (0, 0)),
        ],
        out_specs=[
            pl.BlockSpec((MT1, C), lambda h, m: (m, 0)),
            pl.BlockSpec((1, C), lambda h, m: (0, 0)),
            pl.BlockSpec((1, C), lambda h, m: (0, 0)),
        ],
        out_shape=[
            jax.ShapeDtypeStruct((N, C), jnp.float32),
            jax.ShapeDtypeStruct((1, C), jnp.float32),
            jax.ShapeDtypeStruct((1, C), jnp.float32),
        ],
        scratch_shapes=[pltpu.VMEM((N, C), jnp.float32)],
        compiler_params=pltpu.CompilerParams(
            dimension_semantics=("arbitrary", "arbitrary")),
    )(pooled_rois, W1r, b1r)

    MT2 = 2000
    x2, s2, q2 = pl.pallas_call(
        functools.partial(_mid_body, n=float(N)),
        grid=(N // MT2,),
        in_specs=[
            pl.BlockSpec((MT2, C), lambda m: (m, 0)),
            pl.BlockSpec((1, C), lambda m: (0, 0)),
            pl.BlockSpec((1, C), lambda m: (0, 0)),
            pl.BlockSpec((1, C), lambda m: (0, 0)),
            pl.BlockSpec((1, C), lambda m: (0, 0)),
            pl.BlockSpec((C, C), lambda m: (0, 0)),
            pl.BlockSpec((1, C), lambda m: (0, 0)),
        ],
        out_specs=[
            pl.BlockSpec((MT2, C), lambda m: (m, 0)),
            pl.BlockSpec((1, C), lambda m: (0, 0)),
            pl.BlockSpec((1, C), lambda m: (0, 0)),
        ],
        out_shape=[
            jax.ShapeDtypeStruct((N, C), jnp.float32),
            jax.ShapeDtypeStruct((1, C), jnp.float32),
            jax.ShapeDtypeStruct((1, C), jnp.float32),
        ],
        compiler_params=pltpu.CompilerParams(
            dimension_semantics=("arbitrary",)),
    )(x1, s1, q1, g1r, bt1r, W2r, b2r)

    lo, po, do_ = pl.pallas_call(
        functools.partial(_head_body, n=float(N)),
        grid=(N // MT2,),
        in_specs=[
            pl.BlockSpec((MT2, C), lambda m: (m, 0)),
            pl.BlockSpec((1, C), lambda m: (0, 0)),
            pl.BlockSpec((1, C), lambda m: (0, 0)),
            pl.BlockSpec((1, C), lambda m: (0, 0)),
            pl.BlockSpec((1, C), lambda m: (0, 0)),
            pl.BlockSpec((C, LP), lambda m: (0, 0)),
            pl.BlockSpec((1, LP), lambda m: (0, 0)),
            pl.BlockSpec((C, DP), lambda m: (0, 0)),
            pl.BlockSpec((1, DP), lambda m: (0, 0)),
        ],
        out_specs=[
            pl.BlockSpec((MT2, LP), lambda m: (m, 0)),
            pl.BlockSpec((MT2, LP), lambda m: (m, 0)),
            pl.BlockSpec((MT2, DP), lambda m: (m, 0)),
        ],
        out_shape=[
            jax.ShapeDtypeStruct((N, LP), jnp.float32),
            jax.ShapeDtypeStruct((N, LP), jnp.float32),
            jax.ShapeDtypeStruct((N, DP), jnp.float32),
        ],
        compiler_params=pltpu.CompilerParams(
            dimension_semantics=("arbitrary",)),
    )(x2, s2, q2, g2r, bt2r, Wl_p, bl_p, Wd_p, bd_p)

    logits = lo[:, :NCLS]
    probs = po[:, :NCLS]
    deltas = do_[:, :4 * NCLS].reshape(N, NCLS, 4)
    return (logits, probs, deltas)


# speedup vs baseline: 1.0716x; 1.0589x over previous
import functools

import jax
import jax.numpy as jnp
from jax.experimental import pallas as pl
from jax.experimental.pallas import tpu as pltpu

NCLS = 81
BN_EPS = 1e-3
NEG = -1e30


def _gemm1_body(a_ref, w_ref, b_ref, o_ref, s_ref, q_ref, *, mt):
    h = pl.program_id(1)
    kr = w_ref.shape[1]

    a = a_ref[...].reshape(mt, kr).astype(jnp.bfloat16)
    wt = w_ref[0].astype(jnp.bfloat16)
    y = jnp.dot(a, wt, preferred_element_type=jnp.float32)

    @pl.when(h == 0)
    def _init():
        o_ref[...] = y

    @pl.when((h > 0) & (h < 6))
    def _acc():
        o_ref[...] += y

    @pl.when(h == 6)
    def _fini():
        x = o_ref[...] + y + b_ref[...]
        o_ref[...] = x

        @pl.when(pl.program_id(0) == 0)
        def _zero():
            s_ref[...] = jnp.zeros_like(s_ref)
            q_ref[...] = jnp.zeros_like(q_ref)

        s_ref[...] += jnp.sum(x, axis=0, keepdims=True)
        q_ref[...] += jnp.sum(x * x, axis=0, keepdims=True)


def _mid_body(x_ref, s_ref, q_ref, g_ref, bt_ref, w_ref, b_ref,
              o_ref, s2_ref, q2_ref, *, n):
    inv_n = 1.0 / n
    mean = s_ref[...] * inv_n
    var = q_ref[...] * inv_n - mean * mean
    scale = g_ref[...] * jax.lax.rsqrt(var + BN_EPS)
    xn = jnp.maximum((x_ref[...] - mean) * scale + bt_ref[...], 0.0)
    y = jnp.dot(xn.astype(jnp.bfloat16), w_ref[...].astype(jnp.bfloat16),
                preferred_element_type=jnp.float32) + b_ref[...]
    o_ref[...] = y

    @pl.when(pl.program_id(0) == 0)
    def _zero():
        s2_ref[...] = jnp.zeros_like(s2_ref)
        q2_ref[...] = jnp.zeros_like(q2_ref)

    s2_ref[...] += jnp.sum(y, axis=0, keepdims=True)
    q2_ref[...] += jnp.sum(y * y, axis=0, keepdims=True)


def _head_body(x_ref, s_ref, q_ref, g_ref, bt_ref, wl_ref, bl_ref,
               wd_ref, bd_ref, lo_ref, po_ref, do_ref, *, n):
    inv_n = 1.0 / n
    mean = s_ref[...] * inv_n
    var = q_ref[...] * inv_n - mean * mean
    scale = g_ref[...] * jax.lax.rsqrt(var + BN_EPS)
    xn = jnp.maximum((x_ref[...] - mean) * scale + bt_ref[...], 0.0)
    xb = xn.astype(jnp.bfloat16)

    logits = jnp.dot(xb, wl_ref[...].astype(jnp.bfloat16),
                     preferred_element_type=jnp.float32) + bl_ref[...]
    lo_ref[...] = logits
    m = jnp.max(logits, axis=-1, keepdims=True)
    e = jnp.exp(logits - m)
    po_ref[...] = e / jnp.sum(e, axis=-1, keepdims=True)

    do_ref[...] = jnp.dot(xb, wd_ref[...].astype(jnp.bfloat16),
                          preferred_element_type=jnp.float32) + bd_ref[...]


def kernel(pooled_rois, W1, b1, gamma1, beta1, W2, b2, gamma2, beta2, Wl, bl, Wd, bd):
    N = pooled_rois.shape[0]
    K = 7 * 7 * pooled_rois.shape[3]
    C = W1.shape[3]

    W2r = W2.reshape(C, C)

    LP = 128
    DP = 384
    Wl_p = jnp.zeros((C, LP), W1.dtype).at[:, :NCLS].set(Wl)
    bl_p = jnp.full((1, LP), NEG, W1.dtype).at[0, :NCLS].set(bl)
    Wd_p = jnp.zeros((C, DP), W1.dtype).at[:, :4 * NCLS].set(Wd)
    bd_p = jnp.zeros((1, DP), W1.dtype).at[0, :4 * NCLS].set(bd)

    b1r = b1.reshape(1, C)
    b2r = b2.reshape(1, C)
    g1r = gamma1.reshape(1, C)
    bt1r = beta1.reshape(1, C)
    g2r = gamma2.reshape(1, C)
    bt2r = beta2.reshape(1, C)

    CH = pooled_rois.shape[3]
    KR = 7 * CH
    W1r = W1.reshape(7, KR, C)
    MT1 = 1000
    x1, s1, q1 = pl.pallas_call(
        functools.partial(_gemm1_body, mt=MT1),
        grid=(N // MT1, 7),
        in_specs=[
            pl.BlockSpec((MT1, 1, 7, CH), lambda m, h: (m, h, 0, 0)),
            pl.BlockSpec((1, KR, C), lambda m, h: (h, 0, 0)),
            pl.BlockSpec((1, C), lambda m, h: (0, 0)),
        ],
        out_specs=[
            pl.BlockSpec((MT1, C), lambda m, h: (m, 0)),
            pl.BlockSpec((1, C), lambda m, h: (0, 0)),
            pl.BlockSpec((1, C), lambda m, h: (0, 0)),
        ],
        out_shape=[
            jax.ShapeDtypeStruct((N, C), jnp.float32),
            jax.ShapeDtypeStruct((1, C), jnp.float32),
            jax.ShapeDtypeStruct((1, C), jnp.float32),
        ],
        compiler_params=pltpu.CompilerParams(
            dimension_semantics=("arbitrary", "arbitrary")),
    )(pooled_rois, W1r, b1r)

    MT2 = 2000
    x2, s2, q2 = pl.pallas_call(
        functools.partial(_mid_body, n=float(N)),
        grid=(N // MT2,),
        in_specs=[
            pl.BlockSpec((MT2, C), lambda m: (m, 0)),
            pl.BlockSpec((1, C), lambda m: (0, 0)),
            pl.BlockSpec((1, C), lambda m: (0, 0)),
            pl.BlockSpec((1, C), lambda m: (0, 0)),
            pl.BlockSpec((1, C), lambda m: (0, 0)),
            pl.BlockSpec((C, C), lambda m: (0, 0)),
            pl.BlockSpec((1, C), lambda m: (0, 0)),
        ],
        out_specs=[
            pl.BlockSpec((MT2, C), lambda m: (m, 0)),
            pl.BlockSpec((1, C), lambda m: (0, 0)),
            pl.BlockSpec((1, C), lambda m: (0, 0)),
        ],
        out_shape=[
            jax.ShapeDtypeStruct((N, C), jnp.float32),
            jax.ShapeDtypeStruct((1, C), jnp.float32),
            jax.ShapeDtypeStruct((1, C), jnp.float32),
        ],
        compiler_params=pltpu.CompilerParams(
            dimension_semantics=("arbitrary",)),
    )(x1, s1, q1, g1r, bt1r, W2r, b2r)

    lo, po, do_ = pl.pallas_call(
        functools.partial(_head_body, n=float(N)),
        grid=(N // MT2,),
        in_specs=[
            pl.BlockSpec((MT2, C), lambda m: (m, 0)),
            pl.BlockSpec((1, C), lambda m: (0, 0)),
            pl.BlockSpec((1, C), lambda m: (0, 0)),
            pl.BlockSpec((1, C), lambda m: (0, 0)),
            pl.BlockSpec((1, C), lambda m: (0, 0)),
            pl.BlockSpec((C, LP), lambda m: (0, 0)),
            pl.BlockSpec((1, LP), lambda m: (0, 0)),
            pl.BlockSpec((C, DP), lambda m: (0, 0)),
            pl.BlockSpec((1, DP), lambda m: (0, 0)),
        ],
        out_specs=[
            pl.BlockSpec((MT2, LP), lambda m: (m, 0)),
            pl.BlockSpec((MT2, LP), lambda m: (m, 0)),
            pl.BlockSpec((MT2, DP), lambda m: (m, 0)),
        ],
        out_shape=[
            jax.ShapeDtypeStruct((N, LP), jnp.float32),
            jax.ShapeDtypeStruct((N, LP), jnp.float32),
            jax.ShapeDtypeStruct((N, DP), jnp.float32),
        ],
        compiler_params=pltpu.CompilerParams(
            dimension_semantics=("arbitrary",)),
    )(x2, s2, q2, g2r, bt2r, Wl_p, bl_p, Wd_p, bd_p)

    logits = lo[:, :NCLS]
    probs = po[:, :NCLS]
    deltas = do_[:, :4 * NCLS].reshape(N, NCLS, 4)
    return (logits, probs, deltas)
